# Initial kernel scaffold; baseline (speedup 1.0000x reference)
import jax
import jax.numpy as jnp
from jax.experimental import pallas as pl


def kernel(x, edge_index, e_var, params):
    raise NotImplementedError("write your pallas kernel here")


# speedup vs baseline: 1.4705x; 1.4705x over previous
import functools

import jax
import jax.numpy as jnp
from jax import lax
from jax.experimental import pallas as pl
from jax.experimental.pallas import tpu as pltpu
from jax.experimental.pallas import tpu_sc as plsc

_N = 10000
_E = 160000
_F = 256
_CUT = 1.0

_NC = 2
_NS = 16
_NW = _NC * _NS
_CHUNK = 128
_EP = 163840
_CPW = _EP // (_NW * _CHUNK)

_NBLK = 400
_EBLK = 512

_FQ = 64
_NP = 10240
_RPT = _NP // _NS


def _sig(t):
    return 1.0 / (1.0 + jnp.exp(-t))


def _silu(t):
    return t * _sig(t)


def _mm(a, b):
    return lax.dot_general(a, b, (((1,), (0,)), ((), ())),
                           preferred_element_type=jnp.float32)


def _mlp2(xb, w1, b1, w2, b2):
    h = _silu(_mm(xb, w1) + b1)
    return _mm(h, w2) + b2


def _node_body(xb, wk1, bk1, wk2, bk2, wq1, bq1, wq2, bq2,
               wv1, bv1, wv2, bv2, ws11, bs11, ws12, bs12,
               ws21, bs21, ws22, bs22, ko, qo, vo, s1o, s2o):
    x = xb[...]
    ko[...] = _mlp2(x, wk1[...], bk1[...], wk2[...], bk2[...])
    qo[...] = _mlp2(x, wq1[...], bq1[...], wq2[...], bq2[...])
    v = _mlp2(x, wv1[...], bv1[...], wv2[...], bv2[...])
    vo[...] = v
    s1o[...] = _mlp2(v, ws11[...], bs11[...], ws12[...], bs12[...])
    s2o[...] = _mlp2(v, ws21[...], bs21[...], ws22[...], bs22[...])


def _node_mlps(x, params):
    n = x.shape[0]
    grid = (n // _NBLK,)
    xspec = pl.BlockSpec((_NBLK, _F), lambda i: (i, 0))
    wspec = pl.BlockSpec((_F, _F), lambda i: (0, 0))
    bspec = pl.BlockSpec((1, _F), lambda i: (0, 0))
    ospec = pl.BlockSpec((_NBLK, _F), lambda i: (i, 0))
    args = []
    for name in ('K', 'Q', 'V', 'S1', 'S2'):
        p = params[name]
        args += [p['W1'], p['b1'].reshape(1, _F), p['W2'], p['b2'].reshape(1, _F)]
    in_specs = [xspec] + [wspec, bspec, wspec, bspec] * 5
    out = jax.ShapeDtypeStruct((n, _F), jnp.float32)
    return pl.pallas_call(
        _node_body, grid=grid, in_specs=in_specs,
        out_specs=[ospec] * 5, out_shape=[out] * 5,
    )(x, *args)


def _gather_body(k_hbm, q_hbm, v_hbm, src_hbm, dst_hbm,
                 kg_hbm, qg_hbm, vg_hbm, idx_v, rows_v, sem):
    wid = lax.axis_index("s") * _NC + lax.axis_index("c")

    def step(t, carry):
        r = wid * _CPW + t
        base = r * _CHUNK
        pltpu.sync_copy(src_hbm.at[r], idx_v)
        pltpu.async_copy(k_hbm.at[idx_v], rows_v, sem).wait()
        pltpu.sync_copy(rows_v, kg_hbm.at[pl.ds(base, _CHUNK)])
        pltpu.async_copy(v_hbm.at[idx_v], rows_v, sem).wait()
        pltpu.sync_copy(rows_v, vg_hbm.at[pl.ds(base, _CHUNK)])
        pltpu.sync_copy(dst_hbm.at[r], idx_v)
        pltpu.async_copy(q_hbm.at[idx_v], rows_v, sem).wait()
        pltpu.sync_copy(rows_v, qg_hbm.at[pl.ds(base, _CHUNK)])
        return carry

    lax.fori_loop(0, _CPW, step, 0)


def _gather3(k, q, v, src2d, dst2d):
    mesh = plsc.VectorSubcoreMesh(core_axis_name="c", subcore_axis_name="s",
                                  num_cores=_NC, num_subcores=_NS)
    out = jax.ShapeDtypeStruct((_EP, _F), jnp.float32)
    fn = pl.kernel(
        _gather_body, out_type=[out, out, out], mesh=mesh,
        scratch_types=[
            pltpu.VMEM((_CHUNK,), jnp.int32),
            pltpu.VMEM((_CHUNK, _F), jnp.float32),
            pltpu.SemaphoreType.DMA,
        ],
    )
    return fn(k, q, v, src2d, dst2d)


def _edge_body(ev, kg, qg, vg, cen, wk1, bk1, wk2, bk2,
               wv1, bv1, wv2, bv2, mo):
    i = pl.program_id(0)
    d = ev[0, 0]
    cut = jnp.where(d < _CUT, 0.5 * (jnp.cos(jnp.pi * d / _CUT) + 1.0), 0.0)
    gamma = jnp.float32(_F) / _CUT
    diff = d[:, None] - cen[...]
    bf = jnp.exp(-gamma * diff * diff)
    dk = _silu(_mlp2(bf, wk1[...], bk1[...], wk2[...], bk2[...]) * cut[:, None])
    dv = _silu(_mlp2(bf, wv1[...], bv1[...], wv2[...], bv2[...]) * cut[:, None])
    ke = kg[...] * dk
    wdot = jnp.sum(ke * qg[...], axis=-1)
    weight = _silu(wdot) * cut * (1.0 / jnp.sqrt(jnp.float32(_F)))
    h = vg[...] * dv * weight[:, None]
    eid = i * _EBLK + lax.broadcasted_iota(jnp.int32, (_EBLK, 1), 0)
    valid = eid < _E
    logh = jnp.where(valid, jnp.log(jnp.abs(h)), 0.0)
    sgn = jnp.where(valid & (h < 0), 1.0, 0.0)
    parts = []
    for b in range(4):
        parts.append(logh[:, b * _FQ:(b + 1) * _FQ])
        parts.append(sgn[:, b * _FQ:(b + 1) * _FQ])
    mo[...] = jnp.concatenate(parts, axis=1)


def _edge_stage(ev2d, kg, qg, vg, centers, params):
    grid = (_EP // _EBLK,)
    espec = pl.BlockSpec((1, 1, _EBLK), lambda i: (i, 0, 0))
    gspec = pl.BlockSpec((_EBLK, _F), lambda i: (i, 0))
    cspec = pl.BlockSpec((1, _F), lambda i: (0, 0))
    wspec = pl.BlockSpec((_F, _F), lambda i: (0, 0))
    bspec = pl.BlockSpec((1, _F), lambda i: (0, 0))
    args = []
    for name in ('dK', 'dV'):
        p = params[name]
        args += [p['W1'], p['b1'].reshape(1, _F), p['W2'], p['b2'].reshape(1, _F)]
    out = jax.ShapeDtypeStruct((_EP, 2 * _F), jnp.float32)
    ospec = pl.BlockSpec((_EBLK, 2 * _F), lambda i: (i, 0))
    return pl.pallas_call(
        _edge_body, grid=grid,
        in_specs=[espec, gspec, gspec, gspec, cspec] + [wspec, bspec] * 4,
        out_specs=ospec, out_shape=out,
    )(ev2d, kg, qg, vg, centers.reshape(1, _F), *args)


def _scatter_body(m_hbm, dst_hbm, zero_hbm, agg_hbm, idx_v, val_v, acc):
    cid = lax.axis_index("c")
    sid = lax.axis_index("s")
    row0 = sid * _RPT
    cpt = _EP // _CHUNK // _NS

    for p in range(2):
        f0 = (cid * 2 + p) * 2 * _FQ
        pltpu.sync_copy(zero_hbm.at[pl.ds(row0, _RPT)], acc.at[pl.ds(row0, _RPT)])
        plsc.subcore_barrier()

        def step(t, carry):
            r = sid * cpt + t
            base = r * _CHUNK
            pltpu.sync_copy(dst_hbm.at[r], idx_v)
            pltpu.sync_copy(m_hbm.at[pl.ds(base, _CHUNK), pl.ds(f0, 2 * _FQ)],
                            val_v)
            pltpu.sync_copy(val_v, acc.at[idx_v], add=True)
            return carry

        lax.fori_loop(0, cpt, step, 0)
        plsc.subcore_barrier()
        pltpu.sync_copy(acc.at[pl.ds(row0, _RPT)],
                        agg_hbm.at[pl.ds(row0, _RPT), pl.ds(f0, 2 * _FQ)])
        plsc.subcore_barrier()


def _scatter2(m, dst2d, zeros):
    mesh = plsc.VectorSubcoreMesh(core_axis_name="c", subcore_axis_name="s",
                                  num_cores=_NC, num_subcores=_NS)
    out = jax.ShapeDtypeStruct((_NP, 2 * _F), jnp.float32)
    fn = pl.kernel(
        _scatter_body, out_type=out, mesh=mesh,
        scratch_types=[
            pltpu.VMEM((_CHUNK,), jnp.int32),
            pltpu.VMEM((_CHUNK, 2 * _FQ), jnp.float32),
            pltpu.VMEM_SHARED((_NP, 2 * _FQ), jnp.float32),
        ],
    )
    return fn(m, dst2d, zeros)


def _out_body(mb, w1, b1, w2, b2, yo):
    m = mb[...]
    lparts, sparts = [], []
    for b in range(4):
        lparts.append(m[:, (2 * b) * _FQ:(2 * b + 1) * _FQ])
        sparts.append(m[:, (2 * b + 1) * _FQ:(2 * b + 2) * _FQ])
    lagg = jnp.concatenate(lparts, axis=1)
    nagg = jnp.concatenate(sparts, axis=1)
    odd = jnp.mod(nagg, 2.0)
    hagg = (1.0 - 2.0 * odd) * jnp.exp(lagg)
    yo[...] = _mlp2(hagg, w1[...], b1[...], w2[...], b2[...])


def _out_mlp(agg, params):
    grid = (_NP // 512,)
    mspec = pl.BlockSpec((512, 2 * _F), lambda i: (i, 0))
    spec = pl.BlockSpec((512, _F), lambda i: (i, 0))
    wspec = pl.BlockSpec((_F, _F), lambda i: (0, 0))
    bspec = pl.BlockSpec((1, _F), lambda i: (0, 0))
    p = params['IB']
    return pl.pallas_call(
        _out_body, grid=grid,
        in_specs=[mspec, wspec, bspec, wspec, bspec],
        out_specs=spec, out_shape=jax.ShapeDtypeStruct((_NP, _F), jnp.float32),
    )(agg, p['W1'], p['b1'].reshape(1, _F), p['W2'], p['b2'].reshape(1, _F))


def kernel(x, edge_index, e_var, params):
    src = edge_index[0].astype(jnp.int32)
    dst = edge_index[1].astype(jnp.int32)
    pad = _EP - _E
    src2d = jnp.pad(src, (0, pad)).reshape(_EP // _CHUNK, _CHUNK)
    dst2d = jnp.pad(dst, (0, pad)).reshape(_EP // _CHUNK, _CHUNK)
    ev2d = jnp.pad(e_var, (0, pad)).reshape(_EP // _EBLK, 1, _EBLK)
    centers = jnp.linspace(0.0, _CUT, _F, dtype=jnp.float32)
    zeros = jnp.zeros((_NP, 2 * _FQ), jnp.float32)

    k, q, v, s1, s2 = _node_mlps(x, params)
    kg, qg, vg = _gather3(k, q, v, src2d, dst2d)
    m = _edge_stage(ev2d, kg, qg, vg, centers, params)
    agg = _scatter2(m, dst2d, zeros)
    y = _out_mlp(agg, params)
    return (y[:_N], s1, s2)


# speedup vs baseline: 1.6705x; 1.1360x over previous
import functools

import jax
import jax.numpy as jnp
from jax import lax
from jax.experimental import pallas as pl
from jax.experimental.pallas import tpu as pltpu
from jax.experimental.pallas import tpu_sc as plsc

_N = 10000
_E = 160000
_F = 256
_CUT = 1.0

_NC = 2
_NS = 16
_NW = _NC * _NS
_CHUNK = 128
_EP = 163840
_CPW = _EP // (_NW * _CHUNK)

_NBLK = 400
_EBLK = 512

_FQ = 64
_NP = 10240
_RPT = _NP // _NS


def _sig(t):
    return 1.0 / (1.0 + jnp.exp(-t))


def _silu(t):
    return t * _sig(t)


def _mm(a, b):
    return lax.dot_general(a, b, (((1,), (0,)), ((), ())),
                           preferred_element_type=jnp.float32)


def _mlp2(xb, w1, b1, w2, b2):
    h = _silu(_mm(xb, w1) + b1)
    return _mm(h, w2) + b2


def _node_body(xb, wk1, bk1, wk2, bk2, wq1, bq1, wq2, bq2,
               wv1, bv1, wv2, bv2, ws11, bs11, ws12, bs12,
               ws21, bs21, ws22, bs22, ko, qo, vo, s1o, s2o):
    x = xb[...]
    ko[...] = _mlp2(x, wk1[...], bk1[...], wk2[...], bk2[...])
    qo[...] = _mlp2(x, wq1[...], bq1[...], wq2[...], bq2[...])
    v = _mlp2(x, wv1[...], bv1[...], wv2[...], bv2[...])
    vo[...] = v
    s1o[...] = _mlp2(v, ws11[...], bs11[...], ws12[...], bs12[...])
    s2o[...] = _mlp2(v, ws21[...], bs21[...], ws22[...], bs22[...])


def _node_mlps(x, params):
    n = x.shape[0]
    grid = (n // _NBLK,)
    xspec = pl.BlockSpec((_NBLK, _F), lambda i: (i, 0))
    wspec = pl.BlockSpec((_F, _F), lambda i: (0, 0))
    bspec = pl.BlockSpec((1, _F), lambda i: (0, 0))
    ospec = pl.BlockSpec((_NBLK, _F), lambda i: (i, 0))
    args = []
    for name in ('K', 'Q', 'V', 'S1', 'S2'):
        p = params[name]
        args += [p['W1'], p['b1'].reshape(1, _F), p['W2'], p['b2'].reshape(1, _F)]
    in_specs = [xspec] + [wspec, bspec, wspec, bspec] * 5
    out = jax.ShapeDtypeStruct((n, _F), jnp.float32)
    return pl.pallas_call(
        _node_body, grid=grid, in_specs=in_specs,
        out_specs=[ospec] * 5, out_shape=[out] * 5,
    )(x, *args)


def _gather_phase(table_hbm, out_hbm, idx_v, rows2, wid, g0, g1, w0, w1):
    half = _CPW // 2

    def gath(t, rbuf, sem):
        pltpu.async_copy(table_hbm.at[idx_v.at[t]], rbuf, sem)

    def wb(t, rbuf, sem):
        pltpu.async_copy(rbuf, out_hbm.at[pl.ds((wid * _CPW + t) * _CHUNK,
                                                _CHUNK)], sem)

    def gwait(rbuf, sem):
        pltpu.make_async_copy(table_hbm.at[idx_v.at[0]], rbuf, sem).wait()

    def wwait(rbuf, sem):
        pltpu.make_async_copy(rbuf, out_hbm.at[pl.ds(0, _CHUNK)], sem).wait()

    b0 = rows2.at[0]
    b1 = rows2.at[1]
    gath(0, b0, g0)
    gath(1, b1, g1)

    def step(i, carry):
        t0 = 2 * i
        gwait(b0, g0)
        wb(t0, b0, w0)
        gwait(b1, g1)
        wb(t0 + 1, b1, w1)

        @pl.when(i < half - 1)
        def _():
            wwait(b0, w0)
            gath(t0 + 2, b0, g0)
            wwait(b1, w1)
            gath(t0 + 3, b1, g1)

        return carry

    lax.fori_loop(0, half, step, 0)
    wwait(b0, w0)
    wwait(b1, w1)


def _gather_body(k_hbm, q_hbm, v_hbm, src_hbm, dst_hbm,
                 kg_hbm, qg_hbm, vg_hbm, sidx_v, didx_v, rows2, isem,
                 g0, g1, w0, w1):
    wid = lax.axis_index("s") * _NC + lax.axis_index("c")
    pltpu.async_copy(src_hbm.at[pl.ds(wid * _CPW, _CPW)], sidx_v, isem)
    pltpu.async_copy(dst_hbm.at[pl.ds(wid * _CPW, _CPW)], didx_v, isem)
    pltpu.make_async_copy(src_hbm.at[pl.ds(0, _CPW)], sidx_v, isem).wait()
    pltpu.make_async_copy(dst_hbm.at[pl.ds(0, _CPW)], didx_v, isem).wait()
    _gather_phase(k_hbm, kg_hbm, sidx_v, rows2, wid, g0, g1, w0, w1)
    _gather_phase(v_hbm, vg_hbm, sidx_v, rows2, wid, g0, g1, w0, w1)
    _gather_phase(q_hbm, qg_hbm, didx_v, rows2, wid, g0, g1, w0, w1)


def _gather3(k, q, v, src2d, dst2d):
    mesh = plsc.VectorSubcoreMesh(core_axis_name="c", subcore_axis_name="s",
                                  num_cores=_NC, num_subcores=_NS)
    out = jax.ShapeDtypeStruct((_EP, _F), jnp.float32)
    fn = pl.kernel(
        _gather_body, out_type=[out, out, out], mesh=mesh,
        scratch_types=[
            pltpu.VMEM((_CPW, _CHUNK), jnp.int32),
            pltpu.VMEM((_CPW, _CHUNK), jnp.int32),
            pltpu.VMEM((2, _CHUNK, _F), jnp.float32),
            pltpu.SemaphoreType.DMA,
            pltpu.SemaphoreType.DMA,
            pltpu.SemaphoreType.DMA,
            pltpu.SemaphoreType.DMA,
            pltpu.SemaphoreType.DMA,
        ],
    )
    return fn(k, q, v, src2d, dst2d)


def _edge_body(ev, kg, qg, vg, cen, wk1, bk1, wk2, bk2,
               wv1, bv1, wv2, bv2, mo):
    i = pl.program_id(0)
    d = ev[0, 0]
    cut = jnp.where(d < _CUT, 0.5 * (jnp.cos(jnp.pi * d / _CUT) + 1.0), 0.0)
    gamma = jnp.float32(_F) / _CUT
    diff = d[:, None] - cen[...]
    bf = jnp.exp(-gamma * diff * diff)
    dk = _silu(_mlp2(bf, wk1[...], bk1[...], wk2[...], bk2[...]) * cut[:, None])
    dv = _silu(_mlp2(bf, wv1[...], bv1[...], wv2[...], bv2[...]) * cut[:, None])
    ke = kg[...] * dk
    wdot = jnp.sum(ke * qg[...], axis=-1)
    weight = _silu(wdot) * cut * (1.0 / jnp.sqrt(jnp.float32(_F)))
    h = vg[...] * dv * weight[:, None]
    eid = i * _EBLK + lax.broadcasted_iota(jnp.int32, (_EBLK, 1), 0)
    valid = eid < _E
    logh = jnp.where(valid, jnp.log(jnp.abs(h)), 0.0)
    sgn = jnp.where(valid & (h < 0), 1.0, 0.0)
    parts = []
    for b in range(4):
        parts.append(logh[:, b * _FQ:(b + 1) * _FQ])
        parts.append(sgn[:, b * _FQ:(b + 1) * _FQ])
    mo[...] = jnp.concatenate(parts, axis=1)


def _edge_stage(ev2d, kg, qg, vg, centers, params):
    grid = (_EP // _EBLK,)
    espec = pl.BlockSpec((1, 1, _EBLK), lambda i: (i, 0, 0))
    gspec = pl.BlockSpec((_EBLK, _F), lambda i: (i, 0))
    cspec = pl.BlockSpec((1, _F), lambda i: (0, 0))
    wspec = pl.BlockSpec((_F, _F), lambda i: (0, 0))
    bspec = pl.BlockSpec((1, _F), lambda i: (0, 0))
    args = []
    for name in ('dK', 'dV'):
        p = params[name]
        args += [p['W1'], p['b1'].reshape(1, _F), p['W2'], p['b2'].reshape(1, _F)]
    out = jax.ShapeDtypeStruct((_EP, 2 * _F), jnp.float32)
    ospec = pl.BlockSpec((_EBLK, 2 * _F), lambda i: (i, 0))
    return pl.pallas_call(
        _edge_body, grid=grid,
        in_specs=[espec, gspec, gspec, gspec, cspec] + [wspec, bspec] * 4,
        out_specs=ospec, out_shape=out,
    )(ev2d, kg, qg, vg, centers.reshape(1, _F), *args)


def _scatter_body(m_hbm, dst_hbm, zero_hbm, agg_hbm, idx_v, val2, acc,
                  isem, g0, g1):
    cid = lax.axis_index("c")
    sid = lax.axis_index("s")
    row0 = sid * _RPT
    cpt = _EP // _CHUNK // _NS
    half = cpt // 2
    b0 = val2.at[0]
    b1 = val2.at[1]

    pltpu.async_copy(dst_hbm.at[pl.ds(sid * cpt, cpt)], idx_v, isem)
    pltpu.make_async_copy(dst_hbm.at[pl.ds(0, cpt)], idx_v, isem).wait()

    for p in range(2):
        f0 = (cid * 2 + p) * 2 * _FQ
        pltpu.sync_copy(zero_hbm.at[pl.ds(row0, _RPT)], acc.at[pl.ds(row0, _RPT)])
        plsc.subcore_barrier()

        def load(t, rbuf, sem):
            base = (sid * cpt + t) * _CHUNK
            pltpu.async_copy(
                m_hbm.at[pl.ds(base, _CHUNK), pl.ds(f0, 2 * _FQ)], rbuf, sem)

        def lwait(rbuf, sem):
            pltpu.make_async_copy(
                m_hbm.at[pl.ds(0, _CHUNK), pl.ds(f0, 2 * _FQ)], rbuf,
                sem).wait()

        load(0, b0, g0)
        load(1, b1, g1)

        def step(i, carry):
            t0 = 2 * i
            lwait(b0, g0)
            pltpu.sync_copy(b0, acc.at[idx_v.at[t0]], add=True)
            lwait(b1, g1)
            pltpu.sync_copy(b1, acc.at[idx_v.at[t0 + 1]], add=True)

            @pl.when(i < half - 1)
            def _():
                load(t0 + 2, b0, g0)
                load(t0 + 3, b1, g1)

            return carry

        lax.fori_loop(0, half, step, 0)
        plsc.subcore_barrier()
        pltpu.sync_copy(acc.at[pl.ds(row0, _RPT)],
                        agg_hbm.at[pl.ds(row0, _RPT), pl.ds(f0, 2 * _FQ)])
        plsc.subcore_barrier()


def _scatter2(m, dst2d, zeros):
    mesh = plsc.VectorSubcoreMesh(core_axis_name="c", subcore_axis_name="s",
                                  num_cores=_NC, num_subcores=_NS)
    out = jax.ShapeDtypeStruct((_NP, 2 * _F), jnp.float32)
    cpt = _EP // _CHUNK // _NS
    fn = pl.kernel(
        _scatter_body, out_type=out, mesh=mesh,
        scratch_types=[
            pltpu.VMEM((cpt, _CHUNK), jnp.int32),
            pltpu.VMEM((2, _CHUNK, 2 * _FQ), jnp.float32),
            pltpu.VMEM_SHARED((_NP, 2 * _FQ), jnp.float32),
            pltpu.SemaphoreType.DMA,
            pltpu.SemaphoreType.DMA,
            pltpu.SemaphoreType.DMA,
        ],
    )
    return fn(m, dst2d, zeros)


def _out_body(mb, w1, b1, w2, b2, yo):
    m = mb[...]
    lparts, sparts = [], []
    for b in range(4):
        lparts.append(m[:, (2 * b) * _FQ:(2 * b + 1) * _FQ])
        sparts.append(m[:, (2 * b + 1) * _FQ:(2 * b + 2) * _FQ])
    lagg = jnp.concatenate(lparts, axis=1)
    nagg = jnp.concatenate(sparts, axis=1)
    odd = jnp.mod(nagg, 2.0)
    hagg = (1.0 - 2.0 * odd) * jnp.exp(lagg)
    yo[...] = _mlp2(hagg, w1[...], b1[...], w2[...], b2[...])


def _out_mlp(agg, params):
    grid = (_NP // 512,)
    mspec = pl.BlockSpec((512, 2 * _F), lambda i: (i, 0))
    spec = pl.BlockSpec((512, _F), lambda i: (i, 0))
    wspec = pl.BlockSpec((_F, _F), lambda i: (0, 0))
    bspec = pl.BlockSpec((1, _F), lambda i: (0, 0))
    p = params['IB']
    return pl.pallas_call(
        _out_body, grid=grid,
        in_specs=[mspec, wspec, bspec, wspec, bspec],
        out_specs=spec, out_shape=jax.ShapeDtypeStruct((_NP, _F), jnp.float32),
    )(agg, p['W1'], p['b1'].reshape(1, _F), p['W2'], p['b2'].reshape(1, _F))


def kernel(x, edge_index, e_var, params):
    src = edge_index[0].astype(jnp.int32)
    dst = edge_index[1].astype(jnp.int32)
    pad = _EP - _E
    src2d = jnp.pad(src, (0, pad)).reshape(_EP // _CHUNK, _CHUNK)
    dst2d = jnp.pad(dst, (0, pad)).reshape(_EP // _CHUNK, _CHUNK)
    ev2d = jnp.pad(e_var, (0, pad)).reshape(_EP // _EBLK, 1, _EBLK)
    centers = jnp.linspace(0.0, _CUT, _F, dtype=jnp.float32)
    zeros = jnp.zeros((_NP, 2 * _FQ), jnp.float32)

    k, q, v, s1, s2 = _node_mlps(x, params)
    kg, qg, vg = _gather3(k, q, v, src2d, dst2d)
    m = _edge_stage(ev2d, kg, qg, vg, centers, params)
    agg = _scatter2(m, dst2d, zeros)
    y = _out_mlp(agg, params)
    return (y[:_N], s1, s2)


# speedup vs baseline: 1.7273x; 1.0340x over previous
import functools

import jax
import jax.numpy as jnp
from jax import lax
from jax.experimental import pallas as pl
from jax.experimental.pallas import tpu as pltpu
from jax.experimental.pallas import tpu_sc as plsc

_N = 10000
_E = 160000
_F = 256
_CUT = 1.0

_NC = 2
_NS = 16
_NW = _NC * _NS
_CHUNK = 128
_EP = 163840
_CPW = _EP // (_NW * _CHUNK)

_NBLK = 400
_EBLK = 512

_FQ = 64
_NP = 10240
_RPT = _NP // _NS


def _sig(t):
    return 1.0 / (1.0 + jnp.exp(-t))


def _silu(t):
    return t * _sig(t)


def _mm(a, b):
    return lax.dot_general(a, b, (((1,), (0,)), ((), ())),
                           preferred_element_type=jnp.float32)


def _mlp2(xb, w1, b1, w2, b2):
    h = _silu(_mm(xb, w1) + b1)
    return _mm(h, w2) + b2


def _node_body(xb, wk1, bk1, wk2, bk2, wq1, bq1, wq2, bq2,
               wv1, bv1, wv2, bv2, ws11, bs11, ws12, bs12,
               ws21, bs21, ws22, bs22, ko, qo, vo, s1o, s2o):
    x = xb[...]
    ko[...] = _mlp2(x, wk1[...], bk1[...], wk2[...], bk2[...])
    qo[...] = _mlp2(x, wq1[...], bq1[...], wq2[...], bq2[...])
    v = _mlp2(x, wv1[...], bv1[...], wv2[...], bv2[...])
    vo[...] = v
    s1o[...] = _mlp2(v, ws11[...], bs11[...], ws12[...], bs12[...])
    s2o[...] = _mlp2(v, ws21[...], bs21[...], ws22[...], bs22[...])


def _node_mlps(x, params):
    n = x.shape[0]
    grid = (n // _NBLK,)
    xspec = pl.BlockSpec((_NBLK, _F), lambda i: (i, 0))
    wspec = pl.BlockSpec((_F, _F), lambda i: (0, 0))
    bspec = pl.BlockSpec((1, _F), lambda i: (0, 0))
    ospec = pl.BlockSpec((_NBLK, _F), lambda i: (i, 0))
    args = []
    for name in ('K', 'Q', 'V', 'S1', 'S2'):
        p = params[name]
        args += [p['W1'], p['b1'].reshape(1, _F), p['W2'], p['b2'].reshape(1, _F)]
    in_specs = [xspec] + [wspec, bspec, wspec, bspec] * 5
    out = jax.ShapeDtypeStruct((n, _F), jnp.float32)
    return pl.pallas_call(
        _node_body, grid=grid, in_specs=in_specs,
        out_specs=[ospec] * 5, out_shape=[out] * 5,
    )(x, *args)


def _gather_phase(table_hbm, out_hbm, idx_v, rows2, chunk0, half,
                  g0, g1, w0, w1):
    def gath(t, rbuf, sem):
        pltpu.async_copy(table_hbm.at[idx_v.at[t]], rbuf, sem)

    def wb(t, rbuf, sem):
        pltpu.async_copy(rbuf, out_hbm.at[pl.ds((chunk0 + t) * _CHUNK,
                                                _CHUNK)], sem)

    def gwait(rbuf, sem):
        pltpu.make_async_copy(table_hbm.at[idx_v.at[0]], rbuf, sem).wait()

    def wwait(rbuf, sem):
        pltpu.make_async_copy(rbuf, out_hbm.at[pl.ds(0, _CHUNK)], sem).wait()

    b0 = rows2.at[0]
    b1 = rows2.at[1]
    gath(0, b0, g0)
    gath(1, b1, g1)

    def step(i, carry):
        t0 = 2 * i
        gwait(b0, g0)
        wb(t0, b0, w0)
        gwait(b1, g1)
        wb(t0 + 1, b1, w1)

        @pl.when(i < half - 1)
        def _():
            wwait(b0, w0)
            gath(t0 + 2, b0, g0)
            wwait(b1, w1)
            gath(t0 + 3, b1, g1)

        return carry

    lax.fori_loop(0, half, step, 0)
    wwait(b0, w0)
    wwait(b1, w1)


_CPT_FAST = 56
_CPT_SLOW = 80 - _CPT_FAST


def _gather_body(k_hbm, q_hbm, v_hbm, src_hbm, dst_hbm,
                 kg_hbm, qg_hbm, vg_hbm, sidx_v, didx_v, rows2, isem,
                 g0, g1, w0, w1):
    cid = lax.axis_index("c")
    sid = lax.axis_index("s")
    cpt = _CPT_FAST - (_CPT_FAST - _CPT_SLOW) * cid
    chunk0 = cid * (_NS * _CPT_FAST) + sid * cpt
    half = cpt // 2
    pltpu.async_copy(src_hbm.at[pl.ds(chunk0, _CPT_FAST)], sidx_v, isem)
    pltpu.async_copy(dst_hbm.at[pl.ds(chunk0, _CPT_FAST)], didx_v, isem)
    pltpu.make_async_copy(src_hbm.at[pl.ds(0, _CPT_FAST)], sidx_v, isem).wait()
    pltpu.make_async_copy(dst_hbm.at[pl.ds(0, _CPT_FAST)], didx_v, isem).wait()
    _gather_phase(k_hbm, kg_hbm, sidx_v, rows2, chunk0, half, g0, g1, w0, w1)
    _gather_phase(v_hbm, vg_hbm, sidx_v, rows2, chunk0, half, g0, g1, w0, w1)
    _gather_phase(q_hbm, qg_hbm, didx_v, rows2, chunk0, half, g0, g1, w0, w1)


def _gather3(k, q, v, src2d, dst2d):
    mesh = plsc.VectorSubcoreMesh(core_axis_name="c", subcore_axis_name="s",
                                  num_cores=_NC, num_subcores=_NS)
    out = jax.ShapeDtypeStruct((_EP, _F), jnp.float32)
    npad = _NS * _CPT_FAST + (_NS - 1) * _CPT_SLOW + _CPT_FAST
    src_p = jnp.pad(src2d, ((0, npad - src2d.shape[0]), (0, 0)))
    dst_p = jnp.pad(dst2d, ((0, npad - dst2d.shape[0]), (0, 0)))
    fn = pl.kernel(
        _gather_body, out_type=[out, out, out], mesh=mesh,
        scratch_types=[
            pltpu.VMEM((_CPT_FAST, _CHUNK), jnp.int32),
            pltpu.VMEM((_CPT_FAST, _CHUNK), jnp.int32),
            pltpu.VMEM((2, _CHUNK, _F), jnp.float32),
            pltpu.SemaphoreType.DMA,
            pltpu.SemaphoreType.DMA,
            pltpu.SemaphoreType.DMA,
            pltpu.SemaphoreType.DMA,
            pltpu.SemaphoreType.DMA,
        ],
    )
    return fn(k, q, v, src_p, dst_p)


def _edge_body(ev, kg, qg, vg, cen, wk1, bk1, wk2, bk2,
               wv1, bv1, wv2, bv2, mo):
    i = pl.program_id(0)
    d = ev[0, 0]
    cut = jnp.where(d < _CUT, 0.5 * (jnp.cos(jnp.pi * d / _CUT) + 1.0), 0.0)
    gamma = jnp.float32(_F) / _CUT
    diff = d[:, None] - cen[...]
    bf = jnp.exp(-gamma * diff * diff)
    dk = _silu(_mlp2(bf, wk1[...], bk1[...], wk2[...], bk2[...]) * cut[:, None])
    dv = _silu(_mlp2(bf, wv1[...], bv1[...], wv2[...], bv2[...]) * cut[:, None])
    ke = kg[...] * dk
    wdot = jnp.sum(ke * qg[...], axis=-1)
    weight = _silu(wdot) * cut * (1.0 / jnp.sqrt(jnp.float32(_F)))
    h = vg[...] * dv * weight[:, None]
    eid = i * _EBLK + lax.broadcasted_iota(jnp.int32, (_EBLK, 1), 0)
    valid = eid < _E
    logh = jnp.where(valid, jnp.log(jnp.abs(h)), 0.0)
    sgn = jnp.where(valid & (h < 0), 1.0, 0.0)
    parts = []
    for b in range(4):
        parts.append(logh[:, b * _FQ:(b + 1) * _FQ])
        parts.append(sgn[:, b * _FQ:(b + 1) * _FQ])
    mo[...] = jnp.concatenate(parts, axis=1)


def _edge_stage(ev2d, kg, qg, vg, centers, params):
    grid = (_EP // _EBLK,)
    espec = pl.BlockSpec((1, 1, _EBLK), lambda i: (i, 0, 0))
    gspec = pl.BlockSpec((_EBLK, _F), lambda i: (i, 0))
    cspec = pl.BlockSpec((1, _F), lambda i: (0, 0))
    wspec = pl.BlockSpec((_F, _F), lambda i: (0, 0))
    bspec = pl.BlockSpec((1, _F), lambda i: (0, 0))
    args = []
    for name in ('dK', 'dV'):
        p = params[name]
        args += [p['W1'], p['b1'].reshape(1, _F), p['W2'], p['b2'].reshape(1, _F)]
    out = jax.ShapeDtypeStruct((_EP, 2 * _F), jnp.float32)
    ospec = pl.BlockSpec((_EBLK, 2 * _F), lambda i: (i, 0))
    return pl.pallas_call(
        _edge_body, grid=grid,
        in_specs=[espec, gspec, gspec, gspec, cspec] + [wspec, bspec] * 4,
        out_specs=ospec, out_shape=out,
    )(ev2d, kg, qg, vg, centers.reshape(1, _F), *args)


def _scatter_body(m_hbm, dst_hbm, zero_hbm, agg_hbm, idx_v, val2, acc,
                  isem, g0, g1):
    cid = lax.axis_index("c")
    sid = lax.axis_index("s")
    row0 = sid * _RPT
    cpt = _EP // _CHUNK // _NS
    half = cpt // 2
    b0 = val2.at[0]
    b1 = val2.at[1]

    pltpu.async_copy(dst_hbm.at[pl.ds(sid * cpt, cpt)], idx_v, isem)
    pltpu.make_async_copy(dst_hbm.at[pl.ds(0, cpt)], idx_v, isem).wait()

    for p in range(2):
        f0 = (cid * 2 + p) * 2 * _FQ
        pltpu.sync_copy(zero_hbm.at[pl.ds(row0, _RPT)], acc.at[pl.ds(row0, _RPT)])
        plsc.subcore_barrier()

        def load(t, rbuf, sem):
            base = (sid * cpt + t) * _CHUNK
            pltpu.async_copy(
                m_hbm.at[pl.ds(base, _CHUNK), pl.ds(f0, 2 * _FQ)], rbuf, sem)

        def lwait(rbuf, sem):
            pltpu.make_async_copy(
                m_hbm.at[pl.ds(0, _CHUNK), pl.ds(f0, 2 * _FQ)], rbuf,
                sem).wait()

        load(0, b0, g0)
        load(1, b1, g1)

        def step(i, carry):
            t0 = 2 * i
            lwait(b0, g0)
            pltpu.sync_copy(b0, acc.at[idx_v.at[t0]], add=True)
            lwait(b1, g1)
            pltpu.sync_copy(b1, acc.at[idx_v.at[t0 + 1]], add=True)

            @pl.when(i < half - 1)
            def _():
                load(t0 + 2, b0, g0)
                load(t0 + 3, b1, g1)

            return carry

        lax.fori_loop(0, half, step, 0)
        plsc.subcore_barrier()
        pltpu.sync_copy(acc.at[pl.ds(row0, _RPT)],
                        agg_hbm.at[pl.ds(row0, _RPT), pl.ds(f0, 2 * _FQ)])
        plsc.subcore_barrier()


def _scatter2(m, dst2d, zeros):
    mesh = plsc.VectorSubcoreMesh(core_axis_name="c", subcore_axis_name="s",
                                  num_cores=_NC, num_subcores=_NS)
    out = jax.ShapeDtypeStruct((_NP, 2 * _F), jnp.float32)
    cpt = _EP // _CHUNK // _NS
    fn = pl.kernel(
        _scatter_body, out_type=out, mesh=mesh,
        scratch_types=[
            pltpu.VMEM((cpt, _CHUNK), jnp.int32),
            pltpu.VMEM((2, _CHUNK, 2 * _FQ), jnp.float32),
            pltpu.VMEM_SHARED((_NP, 2 * _FQ), jnp.float32),
            pltpu.SemaphoreType.DMA,
            pltpu.SemaphoreType.DMA,
            pltpu.SemaphoreType.DMA,
        ],
    )
    return fn(m, dst2d, zeros)


def _out_body(mb, w1, b1, w2, b2, yo):
    m = mb[...]
    lparts, sparts = [], []
    for b in range(4):
        lparts.append(m[:, (2 * b) * _FQ:(2 * b + 1) * _FQ])
        sparts.append(m[:, (2 * b + 1) * _FQ:(2 * b + 2) * _FQ])
    lagg = jnp.concatenate(lparts, axis=1)
    nagg = jnp.concatenate(sparts, axis=1)
    odd = jnp.mod(nagg, 2.0)
    hagg = (1.0 - 2.0 * odd) * jnp.exp(lagg)
    yo[...] = _mlp2(hagg, w1[...], b1[...], w2[...], b2[...])


def _out_mlp(agg, params):
    grid = (_NP // 512,)
    mspec = pl.BlockSpec((512, 2 * _F), lambda i: (i, 0))
    spec = pl.BlockSpec((512, _F), lambda i: (i, 0))
    wspec = pl.BlockSpec((_F, _F), lambda i: (0, 0))
    bspec = pl.BlockSpec((1, _F), lambda i: (0, 0))
    p = params['IB']
    return pl.pallas_call(
        _out_body, grid=grid,
        in_specs=[mspec, wspec, bspec, wspec, bspec],
        out_specs=spec, out_shape=jax.ShapeDtypeStruct((_NP, _F), jnp.float32),
    )(agg, p['W1'], p['b1'].reshape(1, _F), p['W2'], p['b2'].reshape(1, _F))


def kernel(x, edge_index, e_var, params):
    src = edge_index[0].astype(jnp.int32)
    dst = edge_index[1].astype(jnp.int32)
    pad = _EP - _E
    src2d = jnp.pad(src, (0, pad)).reshape(_EP // _CHUNK, _CHUNK)
    dst2d = jnp.pad(dst, (0, pad)).reshape(_EP // _CHUNK, _CHUNK)
    ev2d = jnp.pad(e_var, (0, pad)).reshape(_EP // _EBLK, 1, _EBLK)
    centers = jnp.linspace(0.0, _CUT, _F, dtype=jnp.float32)
    zeros = jnp.zeros((_NP, 2 * _FQ), jnp.float32)

    k, q, v, s1, s2 = _node_mlps(x, params)
    kg, qg, vg = _gather3(k, q, v, src2d, dst2d)
    m = _edge_stage(ev2d, kg, qg, vg, centers, params)
    agg = _scatter2(m, dst2d, zeros)
    y = _out_mlp(agg, params)
    return (y[:_N], s1, s2)
